# Initial kernel scaffold; baseline (speedup 1.0000x reference)
import jax
import jax.numpy as jnp
from jax.experimental import pallas as pl


def kernel(flat_obs, edge_index, W1, b1, W2, b2, fcW1, fcb1, fcW2, fcb2):
    raise NotImplementedError("write your pallas kernel here")


# speedup vs baseline: 10.0719x; 10.0719x over previous
import functools

import jax
import jax.numpy as jnp
from jax import lax
from jax.experimental import pallas as pl
from jax.experimental.pallas import tpu as pltpu
from jax.experimental.pallas import tpu_sc as plsc

NODES = 250
COLORS = 40
HID = 128
NODE_F = 4
COLOR_F = 3
N = NODES * COLORS
E = 320000

NC = 2
NS = 16
NW = NC * NS
EPW = E // NW
KB = 128
NBLK = EPW // KB
TAIL = EPW - NBLK * KB
NP = 10112
RPW = NP // NS
HH = HID // 2
EPT = E // NS
NBLK2 = EPT // KB
TAIL2 = EPT - NBLK2 * KB

DEGW = 16


def _sc_mesh():
    return plsc.VectorSubcoreMesh(core_axis_name="c", subcore_axis_name="s")


@functools.partial(
    pl.kernel,
    out_type=jax.ShapeDtypeStruct((2 * NP, DEGW), jnp.float32),
    mesh=_sc_mesh(),
    scratch_types=[
        pltpu.VMEM((KB,), jnp.int32),
        pltpu.VMEM((TAIL,), jnp.int32),
        pltpu.VMEM((KB, DEGW), jnp.float32),
        pltpu.VMEM((TAIL, DEGW), jnp.float32),
        pltpu.VMEM((RPW, DEGW), jnp.float32),
        pltpu.VMEM_SHARED((NP, DEGW), jnp.float32),
    ],
    compiler_params=pltpu.CompilerParams(use_tc_tiling_on_sc=False),
)
def _sc_degree(col_hbm, ones_hbm, zeros_hbm, out_hbm,
               cidx, cidx_t, ones_v, ones_t, wbuf, acc):
    c = lax.axis_index("c")
    s = lax.axis_index("s")
    wid = s * NC + c
    pltpu.sync_copy(ones_hbm, ones_v)
    pltpu.sync_copy(ones_hbm.at[pl.ds(0, TAIL)], ones_t)
    pltpu.sync_copy(zeros_hbm, wbuf)
    pltpu.sync_copy(wbuf, acc.at[pl.ds(s * RPW, RPW)])
    plsc.subcore_barrier()
    base = wid * EPW

    def body(b, carry):
        pltpu.sync_copy(col_hbm.at[pl.ds(base + b * KB, KB)], cidx)
        pltpu.sync_copy(ones_v, acc.at[cidx], add=True)
        return carry

    lax.fori_loop(0, NBLK, body, 0)
    pltpu.sync_copy(col_hbm.at[pl.ds(base + NBLK * KB, TAIL)], cidx_t)
    pltpu.sync_copy(ones_t, acc.at[cidx_t], add=True)
    plsc.subcore_barrier()
    pltpu.sync_copy(acc.at[pl.ds(s * RPW, RPW)], wbuf)
    pltpu.sync_copy(wbuf, out_hbm.at[pl.ds(c * NP + s * RPW, RPW)])


@functools.partial(
    pl.kernel,
    out_type=jax.ShapeDtypeStruct((2 * NP, HH), jnp.float32),
    mesh=_sc_mesh(),
    scratch_types=[
        pltpu.VMEM((KB,), jnp.int32),
        pltpu.VMEM((KB,), jnp.int32),
        pltpu.VMEM((TAIL2,), jnp.int32),
        pltpu.VMEM((TAIL2,), jnp.int32),
        pltpu.VMEM((KB, HH), jnp.float32),
        pltpu.VMEM((TAIL2, HH), jnp.float32),
        pltpu.VMEM((RPW, HH), jnp.float32),
        pltpu.VMEM_SHARED((NP, HH), jnp.float32),
        pltpu.SemaphoreType.DMA,
    ],
    compiler_params=pltpu.CompilerParams(use_tc_tiling_on_sc=False),
)
def _sc_propagate(y_hbm, row_hbm, col_hbm, zeros_hbm, out_hbm,
                  ridx, cidx, ridx_t, cidx_t, rows, rows_t, wbuf, acc, sem):
    c = lax.axis_index("c")
    s = lax.axis_index("s")
    pltpu.sync_copy(zeros_hbm, wbuf)
    pltpu.sync_copy(wbuf, acc.at[pl.ds(s * RPW, RPW)])
    plsc.subcore_barrier()
    base = s * EPT

    def body(b, carry):
        off = base + b * KB
        pltpu.sync_copy(row_hbm.at[pl.ds(off, KB)], ridx)
        pltpu.sync_copy(col_hbm.at[pl.ds(off, KB)], cidx)
        pltpu.async_copy(y_hbm.at[c].at[ridx], rows, sem).wait()
        pltpu.sync_copy(rows, acc.at[cidx], add=True)
        return carry

    lax.fori_loop(0, NBLK2, body, 0)
    off = base + NBLK2 * KB
    pltpu.sync_copy(row_hbm.at[pl.ds(off, TAIL2)], ridx_t)
    pltpu.sync_copy(col_hbm.at[pl.ds(off, TAIL2)], cidx_t)
    pltpu.async_copy(y_hbm.at[c].at[ridx_t], rows_t, sem).wait()
    pltpu.sync_copy(rows_t, acc.at[cidx_t], add=True)
    plsc.subcore_barrier()
    pltpu.sync_copy(acc.at[pl.ds(s * RPW, RPW)], wbuf)
    pltpu.sync_copy(wbuf, out_hbm.at[pl.ds(c * NP + s * RPW, RPW)])


def _tc_head(nf, cf, Wn, Wc, degp):
    def body(nf_ref, cf_ref, wn_ref, wc_ref, degp_ref, y1_ref, dinv_ref):
        deg16 = degp_ref[0] + degp_ref[1]
        deg = deg16[:, 0:1] + 1.0
        dinv = lax.rsqrt(deg)
        np_ = jnp.dot(nf_ref[...], wn_ref[...], preferred_element_type=jnp.float32)
        cp = jnp.dot(cf_ref[...], wc_ref[...], preferred_element_type=jnp.float32)
        xw = (np_[:, None, :] + cp[None, :, :]).reshape(N, HID)
        y = dinv * xw
        y1_ref[0] = y[:, :HH]
        y1_ref[1] = y[:, HH:]
        dinv_ref[...] = dinv

    return pl.pallas_call(
        body,
        out_shape=(
            jax.ShapeDtypeStruct((2, N, HH), jnp.float32),
            jax.ShapeDtypeStruct((N, 1), jnp.float32),
        ),
    )(nf, cf, Wn, Wc, degp)


def _tc_mid(P, y1, b1, W2, dinv):
    def body(P_ref, y1_ref, b1_ref, W2_ref, dinv_ref, y2_ref):
        pm = jnp.concatenate([P_ref[0:N, :], P_ref[NP:NP + N, :]], axis=1)
        y1 = jnp.concatenate([y1_ref[0], y1_ref[1]], axis=1)
        dinv = dinv_ref[...]
        x1 = jnp.maximum(dinv * (pm + y1) + b1_ref[...], 0.0)
        y2 = dinv * jnp.dot(x1, W2_ref[...], preferred_element_type=jnp.float32)
        y2_ref[0] = y2[:, :HH]
        y2_ref[1] = y2[:, HH:]

    return pl.pallas_call(
        body,
        out_shape=jax.ShapeDtypeStruct((2, N, HH), jnp.float32),
    )(P, y1, b1, W2, dinv)


def _tc_out(Q, y2, b2, fcW1, fcb1, fcW2, fcb2, dinv):
    def body(Q_ref, y2_ref, b2_ref, fw1_ref, fb1_ref, fw2_ref, fb2_ref,
             dinv_ref, out_ref):
        qm = jnp.concatenate([Q_ref[0:N, :], Q_ref[NP:NP + N, :]], axis=1)
        y2 = jnp.concatenate([y2_ref[0], y2_ref[1]], axis=1)
        dinv = dinv_ref[...]
        x2 = jnp.maximum(dinv * (qm + y2) + b2_ref[...], 0.0)
        h = jnp.maximum(
            jnp.dot(x2, fw1_ref[...], preferred_element_type=jnp.float32)
            + fb1_ref[...], 0.0)
        out_ref[...] = (
            jnp.dot(h, fw2_ref[...], preferred_element_type=jnp.float32)
            + fb2_ref[...])

    return pl.pallas_call(
        body,
        out_shape=jax.ShapeDtypeStruct((N, 1), jnp.float32),
    )(Q, y2, b2, fcW1, fcb1, fcW2, fcb2, dinv)


def kernel(flat_obs, edge_index, W1, b1, W2, b2, fcW1, fcb1, fcW2, fcb2):
    nf = flat_obs[0, 3:3 + NODES * NODE_F].reshape(NODES, NODE_F)
    cf = flat_obs[0, 3 + NODES * NODE_F:3 + NODES * NODE_F + COLORS * COLOR_F]
    cf = cf.reshape(COLORS, COLOR_F)
    row = edge_index[0].astype(jnp.int32)
    col = edge_index[1].astype(jnp.int32)
    Wn = W1[:NODE_F]
    Wc = W1[NODE_F:]

    ones16 = jnp.ones((KB, DEGW), jnp.float32)
    zeros16 = jnp.zeros((RPW, DEGW), jnp.float32)
    zeros64 = jnp.zeros((RPW, HH), jnp.float32)

    degp = _sc_degree(col, ones16, zeros16).reshape(2, NP, DEGW)[:, :N]
    y1, dinv = _tc_head(nf, cf, Wn, Wc, degp)
    P = _sc_propagate(y1, row, col, zeros64)
    y2 = _tc_mid(P, y1, b1.reshape(1, HID), W2, dinv)
    Q = _sc_propagate(y2, row, col, zeros64)
    q = _tc_out(Q, y2, b2.reshape(1, HID), fcW1, fcb1.reshape(1, HID),
                fcW2, fcb2.reshape(1, 1), dinv)
    return q.reshape(1, N)
